# baseline (speedup 1.0000x reference)
import functools

import jax
import jax.numpy as jnp
from jax import lax
from jax.experimental import pallas as pl
from jax.experimental.pallas import tpu as pltpu
from jax.experimental.pallas import tpu_sc as plsc

EPS = 1e-08
N = 3276800
NUM_GROUPS = 16

SC_N = 1310720
TC_N = N - SC_N

NUM_CORES = 2
NUM_SUBCORES = 16
NW = NUM_CORES * NUM_SUBCORES
PER_W = SC_N // NW
CHUNK = 8192
NCHUNK = PER_W // CHUNK
LANES = 16
UNROLL = 16

BLK = 163840
TC_GRID = TC_N // BLK
SUB = 4096
NSUB = BLK // SUB

_mesh = plsc.VectorSubcoreMesh(core_axis_name="c", subcore_axis_name="s")


@functools.partial(
    pl.kernel,
    mesh=_mesh,
    out_type=jax.ShapeDtypeStruct((SC_N,), jnp.float32),
    compiler_params=pltpu.CompilerParams(needs_layout_passes=False),
    scratch_types=[
        pltpu.VMEM((LANES,), jnp.float32),
        pltpu.VMEM((LANES,), jnp.float32),
        pltpu.VMEM((LANES,), jnp.float32),
        pltpu.VMEM((LANES,), jnp.float32),
        pltpu.VMEM((2, CHUNK), jnp.float32),
        pltpu.VMEM((2, CHUNK), jnp.int32),
        pltpu.VMEM((2, CHUNK), jnp.float32),
        pltpu.SemaphoreType.DMA,
        pltpu.SemaphoreType.DMA,
        pltpu.SemaphoreType.DMA,
        pltpu.SemaphoreType.DMA,
    ],
)
def _range_norm_sc(x_hbm, g_hbm, mins_hbm, maxs_hbm, out_hbm,
                   mins_v, maxs_v, a_v, b_v, x2, g2, o2,
                   sem_in0, sem_in1, sem_out0, sem_out1):
    wid = lax.axis_index("s") * NUM_CORES + lax.axis_index("c")
    base = TC_N + wid * PER_W
    obase = wid * PER_W
    sem_in = (sem_in0, sem_in1)
    sem_out = (sem_out0, sem_out1)

    pltpu.sync_copy(mins_hbm, mins_v)
    pltpu.sync_copy(maxs_hbm, maxs_v)
    m = mins_v[...]
    a = (1.0 - 2.0 * EPS) / (maxs_v[...] - m)
    a_v[...] = a
    b_v[...] = EPS - m * a

    def start_in(ci, slot):
        off = base + ci * CHUNK
        dx = pltpu.async_copy(x_hbm.at[pl.ds(off, CHUNK)], x2.at[slot],
                              sem_in[slot])
        dg = pltpu.async_copy(g_hbm.at[pl.ds(off, CHUNK)], g2.at[slot],
                              sem_in[slot])
        return dx, dg

    pending_out = [None, None]
    cur = start_in(0, 0)
    for ci in range(NCHUNK):
        slot = ci % 2
        nxt = start_in(ci + 1, 1 - slot) if ci + 1 < NCHUNK else None
        cur[0].wait()
        cur[1].wait()
        if pending_out[slot] is not None:
            pending_out[slot].wait()

        @plsc.parallel_loop(0, CHUNK, LANES, unroll=UNROLL)
        def vec_body(e, _slot=slot):
            s = pl.ds(e, LANES)
            idx = g2[_slot, s] - 1
            av = plsc.load_gather(a_v, [idx])
            bv = plsc.load_gather(b_v, [idx])
            o2[_slot, s] = x2[_slot, s] * av + bv

        pending_out[slot] = pltpu.async_copy(
            o2.at[slot], out_hbm.at[pl.ds(obase + ci * CHUNK, CHUNK)],
            sem_out[slot])
        cur = nxt
    for d in pending_out:
        if d is not None:
            d.wait()


def _tc_body(a_ref, b_ref, x_ref, g_ref, o_ref):
    for j in range(NSUB):
        s = pl.ds(j * SUB, SUB)
        x = x_ref[s]
        idx = g_ref[s] - 1
        bit = [(idx & (1 << t)) != 0 for t in range(4)]

        def tree(t_ref):
            vals = [t_ref[k] for k in range(NUM_GROUPS)]
            lvl = [jnp.where(bit[0], vals[2 * k + 1], vals[2 * k])
                   for k in range(8)]
            for t in range(1, 4):
                lvl = [jnp.where(bit[t], lvl[2 * k + 1], lvl[2 * k])
                       for k in range(len(lvl) // 2)]
            return lvl[0]

        o_ref[s] = x * tree(a_ref) + tree(b_ref)


def kernel(x, group, mins, maxs):
    a = (1.0 - 2.0 * EPS) / (maxs - mins)
    b = EPS - mins * a
    sc_out = _range_norm_sc(x, group, mins, maxs)
    tc_out = pl.pallas_call(
        _tc_body,
        grid=(TC_GRID,),
        in_specs=[
            pl.BlockSpec(memory_space=pltpu.SMEM),
            pl.BlockSpec(memory_space=pltpu.SMEM),
            pl.BlockSpec((BLK,), lambda i: (i,)),
            pl.BlockSpec((BLK,), lambda i: (i,)),
        ],
        out_specs=pl.BlockSpec((BLK,), lambda i: (i,)),
        out_shape=jax.ShapeDtypeStruct((N,), jnp.float32),
    )(a, b, x, group)
    return lax.dynamic_update_slice(tc_out, sc_out, (TC_N,))


# speedup vs baseline: 1.0220x; 1.0220x over previous
import functools

import jax
import jax.numpy as jnp
from jax import lax
from jax.experimental import pallas as pl
from jax.experimental.pallas import tpu as pltpu
from jax.experimental.pallas import tpu_sc as plsc

EPS = 1e-08
N = 3276800
NUM_GROUPS = 16

SC_N = 1310720
TC_N = N - SC_N

NUM_CORES = 2
NUM_SUBCORES = 16
NW = NUM_CORES * NUM_SUBCORES
PER_W = SC_N // NW
CHUNK = 10240
NCHUNK = PER_W // CHUNK
LANES = 16
UNROLL = 8

BLK = 163840
TC_GRID = TC_N // BLK
SUB = 4096
NSUB = BLK // SUB

_mesh = plsc.VectorSubcoreMesh(core_axis_name="c", subcore_axis_name="s")


@functools.partial(
    pl.kernel,
    mesh=_mesh,
    out_type=jax.ShapeDtypeStruct((SC_N,), jnp.float32),
    compiler_params=pltpu.CompilerParams(needs_layout_passes=False),
    scratch_types=[
        pltpu.VMEM((LANES,), jnp.float32),
        pltpu.VMEM((LANES,), jnp.float32),
        pltpu.VMEM((LANES,), jnp.float32),
        pltpu.VMEM((LANES,), jnp.float32),
        pltpu.VMEM((2, CHUNK), jnp.float32),
        pltpu.VMEM((2, CHUNK), jnp.int32),
        pltpu.VMEM((2, CHUNK), jnp.float32),
        pltpu.SemaphoreType.DMA,
        pltpu.SemaphoreType.DMA,
        pltpu.SemaphoreType.DMA,
        pltpu.SemaphoreType.DMA,
    ],
)
def _range_norm_sc(x_hbm, g_hbm, mins_hbm, maxs_hbm, out_hbm,
                   mins_v, maxs_v, a_v, b_v, x2, g2, o2,
                   sem_in0, sem_in1, sem_out0, sem_out1):
    wid = lax.axis_index("s") * NUM_CORES + lax.axis_index("c")
    base = TC_N + wid * PER_W
    obase = wid * PER_W
    sem_in = (sem_in0, sem_in1)
    sem_out = (sem_out0, sem_out1)

    pltpu.sync_copy(mins_hbm, mins_v)
    pltpu.sync_copy(maxs_hbm, maxs_v)
    m = mins_v[...]
    a = (1.0 - 2.0 * EPS) / (maxs_v[...] - m)
    a_v[...] = a
    b_v[...] = EPS - m * a

    def start_in(ci, slot):
        off = base + ci * CHUNK
        dx = pltpu.async_copy(x_hbm.at[pl.ds(off, CHUNK)], x2.at[slot],
                              sem_in[slot])
        dg = pltpu.async_copy(g_hbm.at[pl.ds(off, CHUNK)], g2.at[slot],
                              sem_in[slot])
        return dx, dg

    pending_out = [None, None]
    cur = start_in(0, 0)
    for ci in range(NCHUNK):
        slot = ci % 2
        nxt = start_in(ci + 1, 1 - slot) if ci + 1 < NCHUNK else None
        cur[0].wait()
        cur[1].wait()
        if pending_out[slot] is not None:
            pending_out[slot].wait()

        @plsc.parallel_loop(0, CHUNK, LANES, unroll=UNROLL)
        def vec_body(e, _slot=slot):
            s = pl.ds(e, LANES)
            idx = g2[_slot, s] - 1
            av = plsc.load_gather(a_v, [idx])
            bv = plsc.load_gather(b_v, [idx])
            o2[_slot, s] = x2[_slot, s] * av + bv

        pending_out[slot] = pltpu.async_copy(
            o2.at[slot], out_hbm.at[pl.ds(obase + ci * CHUNK, CHUNK)],
            sem_out[slot])
        cur = nxt
    for d in pending_out:
        if d is not None:
            d.wait()


def _tc_body(a_ref, b_ref, x_ref, g_ref, o_ref):
    for j in range(NSUB):
        s = pl.ds(j * SUB, SUB)
        x = x_ref[s]
        idx = g_ref[s] - 1
        bit = [(idx & (1 << t)) != 0 for t in range(4)]

        def tree(t_ref):
            vals = [t_ref[k] for k in range(NUM_GROUPS)]
            lvl = [jnp.where(bit[0], vals[2 * k + 1], vals[2 * k])
                   for k in range(8)]
            for t in range(1, 4):
                lvl = [jnp.where(bit[t], lvl[2 * k + 1], lvl[2 * k])
                       for k in range(len(lvl) // 2)]
            return lvl[0]

        o_ref[s] = x * tree(a_ref) + tree(b_ref)


def kernel(x, group, mins, maxs):
    a = (1.0 - 2.0 * EPS) / (maxs - mins)
    b = EPS - mins * a
    sc_out = _range_norm_sc(x, group, mins, maxs)
    tc_out = pl.pallas_call(
        _tc_body,
        grid=(TC_GRID,),
        in_specs=[
            pl.BlockSpec(memory_space=pltpu.SMEM),
            pl.BlockSpec(memory_space=pltpu.SMEM),
            pl.BlockSpec((BLK,), lambda i: (i,)),
            pl.BlockSpec((BLK,), lambda i: (i,)),
        ],
        out_specs=pl.BlockSpec((BLK,), lambda i: (i,)),
        out_shape=jax.ShapeDtypeStruct((N,), jnp.float32),
    )(a, b, x, group)
    return lax.dynamic_update_slice(tc_out, sc_out, (TC_N,))
